# baseline (speedup 1.0000x reference)
import functools
import math

import jax
import jax.numpy as jnp
from jax import lax
from jax.experimental import pallas as pl
from jax.experimental.pallas import tpu as pltpu
from jax.experimental.pallas import tpu_sc as plsc

N_NODES = 100000
N_EDGES = 3200000
D = 128
D_EDGE = 16
N_HEADS = 2
B = 4096
K = 10
MSG_DIM = 100

R = 256
GRID = B // R


def _dense_body(nf_ref, ngh_ref, nge_ref, nt_ref, et_ref, ef_ref,
                pn_row_ref, pn_col_ref,
                tw_ref, tb_ref,
                wq_ref, wk_a, wk_b, wk_c, wv_a, wv_b, wv_c,
                wo_a, wo_b,
                m1_a, m1_b, m1_c, m1bias, m2_ref, m2bias,
                aw1_a, aw1_b, ab1_ref, aw2_ref, ab2_ref,
                mw_a, mw_b, mw_c, mw_d, mb_ref,
                gwi_ref, gwh_ref, gbi_ref, gbh_ref,
                pos_ref, neg_ref, nm_ref, last_ref,
                vv_ref):
    i = pl.program_id(0)
    f32 = jnp.float32
    scale = f32(1.0 / math.sqrt(D))

    tw = tw_ref[...]
    tb = tb_ref[...]
    cosb = jnp.cos(tb)
    q_const = jnp.dot(cosb, wq_ref[...][D:, :], preferred_element_type=f32)

    et = et_ref[...]

    embs = []
    for s in range(3):
        nf = nf_ref[s]
        q = (jnp.dot(nf, wq_ref[...][:D, :], preferred_element_type=f32)
             + q_const) * scale
        l0 = []
        l1 = []
        for k in range(K):
            ngh = ngh_ref[k, s]
            nge = nge_ref[k, s]
            dt = et - nt_ref[k, s]
            tf = jnp.cos(dt * tw + tb)
            kk = (jnp.dot(ngh, wk_a[...], preferred_element_type=f32)
                  + jnp.dot(nge, wk_b[...], preferred_element_type=f32)
                  + jnp.dot(tf, wk_c[...], preferred_element_type=f32))
            vv = (jnp.dot(ngh, wv_a[...], preferred_element_type=f32)
                  + jnp.dot(nge, wv_b[...], preferred_element_type=f32)
                  + jnp.dot(tf, wv_c[...], preferred_element_type=f32))
            vv_ref[k] = vv
            l0.append(jnp.sum(q[:, :D] * kk[:, :D], axis=1, keepdims=True))
            l1.append(jnp.sum(q[:, D:] * kk[:, D:], axis=1, keepdims=True))
        m0 = functools.reduce(jnp.maximum, l0)
        m1 = functools.reduce(jnp.maximum, l1)
        ao0 = jnp.zeros((R, D), f32)
        ao1 = jnp.zeros((R, D), f32)
        s0 = jnp.zeros((R, 1), f32)
        s1 = jnp.zeros((R, 1), f32)
        for k in range(K):
            vv = vv_ref[k]
            w0 = jnp.exp(l0[k] - m0)
            w1 = jnp.exp(l1[k] - m1)
            ao0 = ao0 + w0 * vv[:, :D]
            ao1 = ao1 + w1 * vv[:, D:]
            s0 = s0 + w0
            s1 = s1 + w1
        ao0 = ao0 / s0
        ao1 = ao1 / s1
        ao = (jnp.dot(ao0, wo_a[...], preferred_element_type=f32)
              + jnp.dot(ao1, wo_b[...], preferred_element_type=f32))
        h1 = (jnp.dot(ao[:, :D], m1_a[...], preferred_element_type=f32)
              + jnp.dot(ao[:, D:], m1_b[...], preferred_element_type=f32)
              + jnp.dot(nf, m1_c[...], preferred_element_type=f32)
              + m1bias[...])
        h1 = jnp.maximum(h1, 0.0)
        emb = jnp.dot(h1, m2_ref[...], preferred_element_type=f32) + m2bias[...]
        embs.append(emb)

    def aff(a, b_):
        x = (jnp.dot(a, aw1_a[...], preferred_element_type=f32)
             + jnp.dot(b_, aw1_b[...], preferred_element_type=f32)
             + ab1_ref[...])
        x = jnp.maximum(x, 0.0)
        y = jnp.dot(x, aw2_ref[...], preferred_element_type=f32) + ab2_ref[...]
        return 1.0 / (1.0 + jnp.exp(-y))
    pos_ref[...] = aff(embs[0], embs[1])
    neg_ref[...] = aff(embs[0], embs[2])

    ef = ef_ref[...]
    tfe = jnp.cos(et * tw + tb)
    nf0 = nf_ref[0]
    nf1 = nf_ref[1]
    for half, (a, b_) in enumerate(((nf0, nf1), (nf1, nf0))):
        msg = (jnp.dot(a, mw_a[...], preferred_element_type=f32)
               + jnp.dot(b_, mw_b[...], preferred_element_type=f32)
               + jnp.dot(ef, mw_c[...], preferred_element_type=f32)
               + jnp.dot(tfe, mw_d[...], preferred_element_type=f32)
               + mb_ref[...])
        msg = jnp.maximum(msg, 0.0)
        gi = jnp.dot(msg, gwi_ref[...], preferred_element_type=f32) + gbi_ref[...]
        gh = jnp.dot(a, gwh_ref[...], preferred_element_type=f32) + gbh_ref[...]
        r = 1.0 / (1.0 + jnp.exp(-(gi[:, :D] + gh[:, :D])))
        z = 1.0 / (1.0 + jnp.exp(-(gi[:, D:2 * D] + gh[:, D:2 * D])))
        g = jnp.tanh(gi[:, 2 * D:] + r * gh[:, 2 * D:])
        nm_ref[half] = (1.0 - z) * g + z * a

    pn_row = pn_row_ref[...]
    for half in range(2):
        mine = pn_col_ref[half]
        acc = jnp.full((R, 1), -1, jnp.int32)
        CH = 1024
        for j0 in range(0, 2 * B, CH):
            chunk = pn_row[:, j0:j0 + CH]
            jidx = lax.broadcasted_iota(jnp.int32, (R, CH), 1) + j0
            eq = mine == chunk
            acc = jnp.maximum(acc, jnp.max(jnp.where(eq, jidx, -1), axis=1,
                                           keepdims=True))
        last_ref[half] = acc


def _dense_call(nf3, ngh4, nge4, nt4, et_col, ef, pn_row, pn_col, params):
    (tw, tb, wq, wk_a, wk_b, wk_c, wv_a, wv_b, wv_c, wo_a, wo_b,
     m1_a, m1_b, m1_c, m1bias, m2, m2bias,
     aw1_a, aw1_b, ab1, aw2, ab2,
     mw_a, mw_b, mw_c, mw_d, mb, gwi, gwh, gbi, gbh) = params
    f32 = jnp.float32
    full = lambda arr: pl.BlockSpec(arr.shape, lambda i: (0,) * arr.ndim)
    in_specs = [
        pl.BlockSpec((3, R, D), lambda i: (0, i, 0)),
        pl.BlockSpec((K, 3, R, D), lambda i: (0, 0, i, 0)),
        pl.BlockSpec((K, 3, R, D_EDGE), lambda i: (0, 0, i, 0)),
        pl.BlockSpec((K, 3, R, 1), lambda i: (0, 0, i, 0)),
        pl.BlockSpec((R, 1), lambda i: (i, 0)),
        pl.BlockSpec((R, D_EDGE), lambda i: (i, 0)),
        full(pn_row),
        pl.BlockSpec((2, R, 1), lambda i: (0, i, 0)),
    ] + [full(p) for p in params]
    out_specs = [
        pl.BlockSpec((R, 1), lambda i: (i, 0)),
        pl.BlockSpec((R, 1), lambda i: (i, 0)),
        pl.BlockSpec((2, R, D), lambda i: (0, i, 0)),
        pl.BlockSpec((2, R, 1), lambda i: (0, i, 0)),
    ]
    out_shape = [
        jax.ShapeDtypeStruct((B, 1), f32),
        jax.ShapeDtypeStruct((B, 1), f32),
        jax.ShapeDtypeStruct((2, B, D), f32),
        jax.ShapeDtypeStruct((2, B, 1), jnp.int32),
    ]
    return pl.pallas_call(
        _dense_body,
        grid=(GRID,),
        in_specs=in_specs,
        out_specs=out_specs,
        out_shape=out_shape,
        scratch_shapes=[pltpu.VMEM((K, R, 2 * D), f32)],
        compiler_params=pltpu.CompilerParams(
            dimension_semantics=("arbitrary",)),
    )(nf3, ngh4, nge4, nt4, et_col, ef, pn_row, pn_col, *params)


NC = 2
NS = 16
NW = NC * NS

NGH_PER_W = 3 * B * K // NW
NGH_CH = 256
NGH_NCH = NGH_PER_W // NGH_CH
NODE_PER_W = 3 * B // NW
NODE_CH = 128
NODE_NCH = NODE_PER_W // NODE_CH
E_PER_W = B // NW
COPY_MAIN = 3120
COPY_CH = 624
COPY_TAIL_BASE = NW * COPY_MAIN

_sc_mesh = lambda: plsc.VectorSubcoreMesh(core_axis_name="c",
                                          subcore_axis_name="s")


def _wid():
    return lax.axis_index("s") * NC + lax.axis_index("c")


def _gather_mem_body(mem_hbm, nghids_hbm, nodes_hbm,
                     nghf_hbm, nodef_hbm,
                     idx_v, nidx_v, rows_v, nrows_v, sem):
    wid = _wid()
    pltpu.sync_copy(nghids_hbm.at[wid], idx_v)
    for c in range(NGH_NCH):
        pltpu.async_copy(mem_hbm.at[idx_v.at[pl.ds(c * NGH_CH, NGH_CH)]],
                         rows_v, sem).wait()
        pltpu.sync_copy(rows_v, nghf_hbm.at[pl.ds(wid * NGH_PER_W + c * NGH_CH,
                                                  NGH_CH)])
    pltpu.sync_copy(nodes_hbm.at[wid], nidx_v)
    for c in range(NODE_NCH):
        pltpu.async_copy(mem_hbm.at[nidx_v.at[pl.ds(c * NODE_CH, NODE_CH)]],
                         nrows_v, sem).wait()
        pltpu.sync_copy(nrows_v, nodef_hbm.at[pl.ds(wid * NODE_PER_W
                                                    + c * NODE_CH, NODE_CH)])


def _gather_edge_body(eft_hbm, ngheid_hbm, eidx_hbm,
                      nghe_hbm, ef_hbm,
                      idx_v, eidx_v, erows_v, efrows_v, sem):
    wid = _wid()
    pltpu.sync_copy(ngheid_hbm.at[wid], idx_v)
    for c in range(NGH_NCH):
        pltpu.async_copy(eft_hbm.at[idx_v.at[pl.ds(c * NGH_CH, NGH_CH)]],
                         erows_v, sem).wait()
        pltpu.sync_copy(erows_v, nghe_hbm.at[pl.ds(wid * NGH_PER_W + c * NGH_CH,
                                                   NGH_CH)])
    pltpu.sync_copy(eidx_hbm.at[wid], eidx_v)
    pltpu.async_copy(eft_hbm.at[eidx_v], efrows_v, sem).wait()
    pltpu.sync_copy(efrows_v, ef_hbm.at[pl.ds(wid * E_PER_W, E_PER_W)])


def _sc_gather(memory, eft, ngh_ids, ngh_eids, nodes, e_idx):
    f32 = jnp.float32
    i32 = jnp.int32
    kmem = pl.kernel(
        _gather_mem_body,
        out_type=(
            jax.ShapeDtypeStruct((3 * B * K, D), f32),
            jax.ShapeDtypeStruct((3 * B, D), f32),
        ),
        mesh=_sc_mesh(),
        scratch_types=[
            pltpu.VMEM((NGH_PER_W,), i32),
            pltpu.VMEM((NODE_PER_W,), i32),
            pltpu.VMEM((NGH_CH, D), f32),
            pltpu.VMEM((NODE_CH, D), f32),
            pltpu.SemaphoreType.DMA,
        ],
        name="tgn_sc_gather_mem",
    )
    kedge = pl.kernel(
        _gather_edge_body,
        out_type=(
            jax.ShapeDtypeStruct((3 * B * K, D_EDGE), f32),
            jax.ShapeDtypeStruct((B, D_EDGE), f32),
        ),
        mesh=_sc_mesh(),
        scratch_types=[
            pltpu.VMEM((NGH_PER_W,), i32),
            pltpu.VMEM((E_PER_W,), i32),
            pltpu.VMEM((NGH_CH, D_EDGE), f32),
            pltpu.VMEM((E_PER_W, D_EDGE), f32),
            pltpu.SemaphoreType.DMA,
        ],
        name="tgn_sc_gather_edge",
        compiler_params=pltpu.CompilerParams(use_tc_tiling_on_sc=False),
    )
    ngh_feat, node_feat = kmem(memory,
                               ngh_ids.reshape(NW, NGH_PER_W),
                               nodes.reshape(NW, NODE_PER_W))
    ngh_edge, e_feat = kedge(eft,
                             ngh_eids.reshape(NW, NGH_PER_W),
                             e_idx.reshape(NW, E_PER_W))
    return ngh_feat, ngh_edge, node_feat, e_feat


def _copy_body(src_hbm, dst_hbm, buf_v, tail_v, sem):
    wid = _wid()
    for c in range(COPY_MAIN // COPY_CH):
        base = wid * COPY_MAIN + c * COPY_CH
        pltpu.async_copy(src_hbm.at[pl.ds(base, COPY_CH)], buf_v, sem).wait()
        pltpu.sync_copy(buf_v, dst_hbm.at[pl.ds(base, COPY_CH)])

    @pl.when(wid < (N_NODES - COPY_TAIL_BASE) // 8)
    def _():
        tbase = COPY_TAIL_BASE + wid * 8
        pltpu.async_copy(src_hbm.at[pl.ds(tbase, 8)], tail_v, sem).wait()
        pltpu.sync_copy(tail_v, dst_hbm.at[pl.ds(tbase, 8)])


def _sc_copy(memory):
    k = pl.kernel(
        _copy_body,
        out_type=jax.ShapeDtypeStruct((N_NODES, D), jnp.float32),
        mesh=_sc_mesh(),
        scratch_types=[pltpu.VMEM((COPY_CH, D), jnp.float32),
                       pltpu.VMEM((8, D), jnp.float32),
                       pltpu.SemaphoreType.DMA],
        name="tgn_sc_copy",
    )
    return k(memory)


def _scatter_body(pos_hbm, last_hbm, newmem_hbm, mem_ref,
                  pidx_v, lidx_v, rows_v, sem):
    wid = _wid()
    pltpu.sync_copy(pos_hbm.at[wid], pidx_v)
    pltpu.sync_copy(last_hbm.at[wid], lidx_v)
    pltpu.async_copy(newmem_hbm.at[lidx_v], rows_v, sem).wait()
    pltpu.async_copy(rows_v, mem_ref.at[pidx_v], sem).wait()


def _sc_scatter(pos2d, last2d, new_mem, mem_ref):
    upd_per_w = 2 * B // NW
    k = pl.kernel(
        _scatter_body,
        out_type=(),
        mesh=_sc_mesh(),
        scratch_types=[
            pltpu.VMEM((upd_per_w,), jnp.int32),
            pltpu.VMEM((upd_per_w,), jnp.int32),
            pltpu.VMEM((upd_per_w, D), jnp.float32),
            pltpu.SemaphoreType.DMA,
        ],
        name="tgn_sc_scatter",
    )
    return k(pos2d, last2d, new_mem, mem_ref)


def _prep_params(time_w, time_b, Wq, Wk, Wv, Wo, merge_w1, merge_b1,
                 merge_w2, merge_b2, aff_w1, aff_b1, aff_w2, aff_b2,
                 msg_w, msg_b, gru_wi, gru_wh, gru_bi, gru_bh):
    row = lambda v: v.reshape(1, -1)
    return (
        row(time_w), row(time_b), Wq,
        Wk[:D], Wk[D:D + D_EDGE], Wk[D + D_EDGE:],
        Wv[:D], Wv[D:D + D_EDGE], Wv[D + D_EDGE:],
        Wo[:D], Wo[D:],
        merge_w1[:D], merge_w1[D:2 * D], merge_w1[2 * D:], row(merge_b1),
        merge_w2, row(merge_b2),
        aff_w1[:D], aff_w1[D:], row(aff_b1), aff_w2, row(aff_b2),
        msg_w[:D], msg_w[D:2 * D], msg_w[2 * D:2 * D + D_EDGE],
        msg_w[2 * D + D_EDGE:], row(msg_b),
        gru_wi, gru_wh, row(gru_bi), row(gru_bh),
    )


def kernel(source_nodes, destination_nodes, negative_nodes, edge_times,
           edge_idxs, neighbor_node_ids, neighbor_edge_idxs, neighbor_times,
           memory, edge_feat_table, time_w, time_b, Wq, Wk, Wv, Wo,
           merge_w1, merge_b1, merge_w2, merge_b2,
           aff_w1, aff_b1, aff_w2, aff_b2,
           msg_w, msg_b, gru_wi, gru_wh, gru_bi, gru_bh):
    nodes = jnp.concatenate([source_nodes, destination_nodes, negative_nodes])
    pos_nodes = nodes[:2 * B]

    ngh_ids_km = neighbor_node_ids.T.reshape(-1)
    ngh_eid_km = neighbor_edge_idxs.T.reshape(-1)
    ngh_feat, ngh_edge, node_feat, e_feat = _sc_gather(
        memory, edge_feat_table, ngh_ids_km, ngh_eid_km, nodes, edge_idxs)

    nf3 = node_feat.reshape(3, B, D)
    ngh4 = ngh_feat.reshape(K, 3, B, D)
    nge4 = ngh_edge.reshape(K, 3, B, D_EDGE)
    nt4 = neighbor_times.T.reshape(K, 3, B, 1)
    et_col = edge_times.reshape(B, 1)
    pn_row = pos_nodes.reshape(1, 2 * B)
    pn_col = pos_nodes.reshape(2, B, 1)
    params = _prep_params(time_w, time_b, Wq, Wk, Wv, Wo, merge_w1, merge_b1,
                          merge_w2, merge_b2, aff_w1, aff_b1, aff_w2, aff_b2,
                          msg_w, msg_b, gru_wi, gru_wh, gru_bi, gru_bh)
    pos2, neg2, nm3, last3 = _dense_call(nf3, ngh4, nge4, nt4, et_col, e_feat,
                                         pn_row, pn_col, params)
    new_mem = nm3.reshape(2 * B, D)

    mem_ref = jax.new_ref(_sc_copy(memory))
    _sc_scatter(pos_nodes.reshape(NW, -1), last3.reshape(NW, -1), new_mem,
                mem_ref)
    updated_memory = mem_ref[...]
    return pos2.reshape(B), neg2.reshape(B), updated_memory
